# baseline (speedup 1.0000x reference)
import functools

import jax
import jax.numpy as jnp
import numpy as np
from jax import lax
from jax.experimental import pallas as pl
from jax.experimental.pallas import tpu as pltpu
from jax.experimental.pallas import tpu_sc as plsc

VOCAB = 100000
EMB = 128
BATCH = 1024
CTX = 20

NC, NS = 2, 16
NW = NC * NS
N_IDX = BATCH * CTX
B_PER_W = N_IDX // NW
ROWS_PER_W = BATCH // NW
ROWS_PER_C = BATCH // NC
IDX_CHUNK = 128
CHUNKS = B_PER_W // IDX_CHUNK
SEG_ROWS = 8

VT = 2000

_SEG_NP = np.zeros((NW, SEG_ROWS, IDX_CHUNK), np.int32)
_seg_flat = ((np.arange(N_IDX) // CTX) % ROWS_PER_C).astype(np.int32)
_SEG_NP[:, :CHUNKS, :] = _seg_flat.reshape(NW, CHUNKS, IDX_CHUNK)


def _sc_embed(table, idx_flat, seg_map, zeros_blk):
    mesh = plsc.VectorSubcoreMesh(core_axis_name="c", subcore_axis_name="s")

    @functools.partial(
        pl.kernel,
        out_type=jax.ShapeDtypeStruct((BATCH, EMB), jnp.float32),
        mesh=mesh,
        scratch_types=[
            pltpu.VMEM((B_PER_W,), jnp.int32),
            pltpu.VMEM((SEG_ROWS, IDX_CHUNK), jnp.int32),
            pltpu.VMEM((B_PER_W, EMB), jnp.float32),
            pltpu.VMEM_SHARED((ROWS_PER_C, EMB), jnp.float32),
            pltpu.SemaphoreType.DMA,
        ],
    )
    def embed_kernel(table_hbm, idx_hbm, seg_hbm, zeros_hbm, out_hbm,
                     idx_v, seg_v, rows_v, emb_sh, sem):
        cid = lax.axis_index("c")
        sid = lax.axis_index("s")
        wid = cid * NS + sid
        pltpu.sync_copy(idx_hbm.at[pl.ds(wid * B_PER_W, B_PER_W)], idx_v)
        pltpu.sync_copy(seg_hbm.at[wid], seg_v)

        @pl.when(sid == 0)
        def _():
            pltpu.sync_copy(zeros_hbm, emb_sh)

        copies = []
        for j in range(CHUNKS):
            copies.append(
                pltpu.async_copy(
                    table_hbm.at[idx_v.at[pl.ds(j * IDX_CHUNK, IDX_CHUNK)]],
                    rows_v.at[pl.ds(j * IDX_CHUNK, IDX_CHUNK)],
                    sem,
                )
            )
        plsc.subcore_barrier()
        for j, c in enumerate(copies):
            c.wait()
            pltpu.sync_copy(
                rows_v.at[pl.ds(j * IDX_CHUNK, IDX_CHUNK)],
                emb_sh.at[seg_v.at[j]],
                add=True,
            )
        plsc.subcore_barrier()
        pltpu.sync_copy(
            emb_sh.at[pl.ds(sid * ROWS_PER_W, ROWS_PER_W)],
            out_hbm.at[pl.ds(wid * ROWS_PER_W, ROWS_PER_W)],
        )

    return embed_kernel(table, idx_flat, seg_map, zeros_blk)


def _tc_project_t(emb, w, bcol):
    mesh = pltpu.create_tensorcore_mesh("core", num_cores=2)

    @functools.partial(
        pl.kernel,
        out_type=jax.ShapeDtypeStruct((VOCAB, BATCH), jnp.float32),
        mesh=mesh,
        scratch_types=[
            pltpu.VMEM((BATCH, EMB), jnp.float32),
            pltpu.VMEM((BATCH, EMB), jnp.bfloat16),
            pltpu.SemaphoreType.DMA,
        ],
    )
    def proj_kernel(emb_hbm, w_hbm, b_hbm, out_hbm, emb_v, ebf_v, sem):
        pltpu.async_copy(emb_hbm, emb_v, sem).wait()
        ebf_v[...] = emb_v[...].astype(jnp.bfloat16)

        def tile_body(w_v, b_v, o_v):
            wt = w_v[...].astype(jnp.bfloat16)
            acc = lax.dot_general(
                wt,
                ebf_v[...],
                dimension_numbers=(((1,), (1,)), ((), ())),
                preferred_element_type=jnp.float32,
            )
            o_v[...] = acc + b_v[...]

        pltpu.emit_pipeline(
            tile_body,
            grid=(VOCAB // VT,),
            in_specs=[
                pl.BlockSpec((VT, EMB), lambda i: (i, 0)),
                pl.BlockSpec((VT, 1), lambda i: (i, 0)),
            ],
            out_specs=[pl.BlockSpec((VT, BATCH), lambda i: (i, 0))],
            core_axis_name="core",
            dimension_semantics=(pltpu.PARALLEL,),
        )(w_hbm, b_hbm, out_hbm)

    return proj_kernel(emb, w, bcol)


def kernel(X, W_emb, W1_w, W1_b):
    seg_map = jnp.asarray(_SEG_NP)
    zeros_blk = jnp.zeros((ROWS_PER_C, EMB), jnp.float32)
    emb = _sc_embed(W_emb, X.reshape(N_IDX), seg_map, zeros_blk)
    out_t = _tc_project_t(emb, W1_w, W1_b.reshape(VOCAB, 1))
    return out_t.T


# speedup vs baseline: 1.0194x; 1.0194x over previous
import functools

import jax
import jax.numpy as jnp
import numpy as np
from jax import lax
from jax.experimental import pallas as pl
from jax.experimental.pallas import tpu as pltpu
from jax.experimental.pallas import tpu_sc as plsc

VOCAB = 100000
EMB = 128
BATCH = 1024
CTX = 20

NC, NS = 2, 16
NW = NC * NS
N_IDX = BATCH * CTX
B_PER_W = N_IDX // NW
ROWS_PER_W = BATCH // NW
ROWS_PER_C = BATCH // NC
IDX_CHUNK = 128
CHUNKS = B_PER_W // IDX_CHUNK
SEG_ROWS = 8

VT = 2000

_SEG_NP = np.zeros((NW, SEG_ROWS, IDX_CHUNK), np.int32)
_seg_flat = ((np.arange(N_IDX) // CTX) % ROWS_PER_C).astype(np.int32)
_SEG_NP[:, :CHUNKS, :] = _seg_flat.reshape(NW, CHUNKS, IDX_CHUNK)


def _sc_embed(table, idx_flat, seg_map, zeros_blk):
    mesh = plsc.VectorSubcoreMesh(core_axis_name="c", subcore_axis_name="s")

    @functools.partial(
        pl.kernel,
        out_type=jax.ShapeDtypeStruct((BATCH, EMB), jnp.float32),
        mesh=mesh,
        scratch_types=[
            pltpu.VMEM((B_PER_W,), jnp.int32),
            pltpu.VMEM((SEG_ROWS, IDX_CHUNK), jnp.int32),
            pltpu.VMEM((B_PER_W, EMB), jnp.float32),
            pltpu.VMEM_SHARED((ROWS_PER_C, EMB), jnp.float32),
            pltpu.SemaphoreType.DMA,
        ],
    )
    def embed_kernel(table_hbm, idx_hbm, seg_hbm, zeros_hbm, out_hbm,
                     idx_v, seg_v, rows_v, emb_sh, sem):
        cid = lax.axis_index("c")
        sid = lax.axis_index("s")
        wid = cid * NS + sid
        pltpu.sync_copy(idx_hbm.at[pl.ds(wid * B_PER_W, B_PER_W)], idx_v)
        pltpu.sync_copy(seg_hbm.at[wid], seg_v)

        @pl.when(sid == 0)
        def _():
            pltpu.sync_copy(zeros_hbm, emb_sh)

        copies = []
        for j in range(CHUNKS):
            copies.append(
                pltpu.async_copy(
                    table_hbm.at[idx_v.at[pl.ds(j * IDX_CHUNK, IDX_CHUNK)]],
                    rows_v.at[pl.ds(j * IDX_CHUNK, IDX_CHUNK)],
                    sem,
                )
            )
        plsc.subcore_barrier()
        for j, c in enumerate(copies):
            c.wait()
            pltpu.sync_copy(
                rows_v.at[pl.ds(j * IDX_CHUNK, IDX_CHUNK)],
                emb_sh.at[seg_v.at[j]],
                add=True,
            )
        plsc.subcore_barrier()
        pltpu.sync_copy(
            emb_sh.at[pl.ds(sid * ROWS_PER_W, ROWS_PER_W)],
            out_hbm.at[pl.ds(wid * ROWS_PER_W, ROWS_PER_W)],
        )

    return embed_kernel(table, idx_flat, seg_map, zeros_blk)


NBUF = 4
NT = VOCAB // VT


def _tc_project_t(emb, w, bcol):
    def body(emb_ref, w_ref, b_ref, o_hbm, ebf_ref, bufs, sems):
        i = pl.program_id(0)

        @pl.when(i == 0)
        def _():
            ebf_ref[...] = emb_ref[...].astype(jnp.bfloat16)

        def out_copy(step, slot):
            return pltpu.make_async_copy(
                bufs.at[slot],
                o_hbm.at[pl.ds(step * VT, VT)],
                sems.at[slot],
            )

        slot = lax.rem(i, NBUF)

        @pl.when(i >= NBUF)
        def _():
            out_copy(i - NBUF, slot).wait()

        wt = w_ref[...].astype(jnp.bfloat16)
        acc = lax.dot_general(
            wt,
            ebf_ref[...],
            dimension_numbers=(((1,), (1,)), ((), ())),
            preferred_element_type=jnp.float32,
        )
        bufs[slot] = acc + b_ref[...]
        out_copy(i, slot).start()

        @pl.when(i == NT - 1)
        def _():
            for j in range(NBUF):
                out_copy(j, j).wait()

    return pl.pallas_call(
        body,
        grid=(NT,),
        in_specs=[
            pl.BlockSpec((BATCH, EMB), lambda i: (0, 0)),
            pl.BlockSpec((VT, EMB), lambda i: (i, 0)),
            pl.BlockSpec((VT, 1), lambda i: (i, 0)),
        ],
        out_specs=pl.BlockSpec(memory_space=pl.ANY),
        out_shape=jax.ShapeDtypeStruct((VOCAB, BATCH), jnp.float32),
        scratch_shapes=[
            pltpu.VMEM((BATCH, EMB), jnp.bfloat16),
            pltpu.VMEM((NBUF, VT, BATCH), jnp.float32),
            pltpu.SemaphoreType.DMA((NBUF,)),
        ],
        compiler_params=pltpu.CompilerParams(
            dimension_semantics=("arbitrary",),
        ),
    )(emb, w, bcol)


def kernel(X, W_emb, W1_w, W1_b):
    seg_map = jnp.asarray(_SEG_NP)
    zeros_blk = jnp.zeros((ROWS_PER_C, EMB), jnp.float32)
    emb = _sc_embed(W_emb, X.reshape(N_IDX), seg_map, zeros_blk)
    out_t = _tc_project_t(emb, W1_w, W1_b.reshape(VOCAB, 1))
    return out_t.T


# speedup vs baseline: 1.1377x; 1.1160x over previous
import functools

import jax
import jax.numpy as jnp
import numpy as np
from jax import lax
from jax.experimental import pallas as pl
from jax.experimental.pallas import tpu as pltpu
from jax.experimental.pallas import tpu_sc as plsc

VOCAB = 100000
EMB = 128
BATCH = 1024
CTX = 20

NC, NS = 2, 16
NW = NC * NS
N_IDX = BATCH * CTX
B_PER_W = N_IDX // NW
ROWS_PER_W = BATCH // NW
ROWS_PER_C = BATCH // NC
IDX_CHUNK = 128
CHUNKS = B_PER_W // IDX_CHUNK
SEG_ROWS = 8

VT = 2048

_SEG_NP = np.zeros((NW, SEG_ROWS, IDX_CHUNK), np.int32)
_seg_flat = ((np.arange(N_IDX) // CTX) % ROWS_PER_C).astype(np.int32)
_SEG_NP[:, :CHUNKS, :] = _seg_flat.reshape(NW, CHUNKS, IDX_CHUNK)


def _sc_embed(table, idx_flat, seg_map, zeros_blk):
    mesh = plsc.VectorSubcoreMesh(core_axis_name="c", subcore_axis_name="s")

    @functools.partial(
        pl.kernel,
        out_type=jax.ShapeDtypeStruct((BATCH, EMB), jnp.float32),
        mesh=mesh,
        scratch_types=[
            pltpu.VMEM((B_PER_W,), jnp.int32),
            pltpu.VMEM((SEG_ROWS, IDX_CHUNK), jnp.int32),
            pltpu.VMEM((B_PER_W, EMB), jnp.float32),
            pltpu.VMEM_SHARED((ROWS_PER_C, EMB), jnp.float32),
            pltpu.SemaphoreType.DMA,
        ],
    )
    def embed_kernel(table_hbm, idx_hbm, seg_hbm, zeros_hbm, out_hbm,
                     idx_v, seg_v, rows_v, emb_sh, sem):
        cid = lax.axis_index("c")
        sid = lax.axis_index("s")
        wid = cid * NS + sid
        pltpu.sync_copy(idx_hbm.at[pl.ds(wid * B_PER_W, B_PER_W)], idx_v)
        pltpu.sync_copy(seg_hbm.at[wid], seg_v)

        @pl.when(sid == 0)
        def _():
            pltpu.sync_copy(zeros_hbm, emb_sh)

        copies = []
        for j in range(CHUNKS):
            copies.append(
                pltpu.async_copy(
                    table_hbm.at[idx_v.at[pl.ds(j * IDX_CHUNK, IDX_CHUNK)]],
                    rows_v.at[pl.ds(j * IDX_CHUNK, IDX_CHUNK)],
                    sem,
                )
            )
        plsc.subcore_barrier()
        for j, c in enumerate(copies):
            c.wait()
            pltpu.sync_copy(
                rows_v.at[pl.ds(j * IDX_CHUNK, IDX_CHUNK)],
                emb_sh.at[seg_v.at[j]],
                add=True,
            )
        plsc.subcore_barrier()
        pltpu.sync_copy(
            emb_sh.at[pl.ds(sid * ROWS_PER_W, ROWS_PER_W)],
            out_hbm.at[pl.ds(wid * ROWS_PER_W, ROWS_PER_W)],
        )

    return embed_kernel(table, idx_flat, seg_map, zeros_blk)


NBUF = 4
GROUP = NBUF * VT
MACRO = -(-VOCAB // GROUP)
VPAD = MACRO * GROUP
LAST_BASE = (MACRO - 1) * GROUP
LAST_ROWS = VOCAB - LAST_BASE


def _tc_project_t(emb, w, bpad):
    def body(emb_ref, w_ref, b_ref, o_hbm, ebf_ref, bufs, sems):
        m = pl.program_id(0)

        @pl.when(m == 0)
        def _():
            ebf_ref[...] = emb_ref[...].astype(jnp.bfloat16)

        def out_copy(macro_idx, j, rows=VT):
            return pltpu.make_async_copy(
                bufs.at[j, pl.ds(0, rows)],
                o_hbm.at[pl.ds(macro_idx * GROUP + j * VT, rows)],
                sems.at[j],
            )

        for j in range(NBUF):
            @pl.when(m >= 1)
            def _(j=j):
                out_copy(m - 1, j).wait()

            wt = w_ref[pl.ds(j * VT, VT), :].astype(jnp.bfloat16)
            acc = lax.dot_general(
                wt,
                ebf_ref[...],
                dimension_numbers=(((1,), (1,)), ((), ())),
                preferred_element_type=jnp.float32,
            )
            bufs[j] = acc + b_ref[pl.ds(j * VT, VT), :]
            if j == 0:
                @pl.when(m < MACRO - 1)
                def _():
                    out_copy(m, 0).start()

                @pl.when(m == MACRO - 1)
                def _():
                    out_copy(MACRO - 1, 0, rows=LAST_ROWS).start()
            else:
                @pl.when(m < MACRO - 1)
                def _(j=j):
                    out_copy(m, j).start()

        @pl.when(m == MACRO - 1)
        def _():
            out_copy(MACRO - 1, 0, rows=LAST_ROWS).wait()

    return pl.pallas_call(
        body,
        grid=(MACRO,),
        in_specs=[
            pl.BlockSpec((BATCH, EMB), lambda i: (0, 0)),
            pl.BlockSpec((GROUP, EMB), lambda i: (i, 0)),
            pl.BlockSpec((GROUP, 1), lambda i: (i, 0)),
        ],
        out_specs=pl.BlockSpec(memory_space=pl.ANY),
        out_shape=jax.ShapeDtypeStruct((VOCAB, BATCH), jnp.float32),
        scratch_shapes=[
            pltpu.VMEM((BATCH, EMB), jnp.bfloat16),
            pltpu.VMEM((NBUF, VT, BATCH), jnp.float32),
            pltpu.SemaphoreType.DMA((NBUF,)),
        ],
        compiler_params=pltpu.CompilerParams(
            dimension_semantics=("arbitrary",),
        ),
    )(emb, w, bpad)


def kernel(X, W_emb, W1_w, W1_b):
    seg_map = jnp.asarray(_SEG_NP)
    zeros_blk = jnp.zeros((ROWS_PER_C, EMB), jnp.float32)
    emb = _sc_embed(W_emb, X.reshape(N_IDX), seg_map, zeros_blk)
    bpad = jnp.pad(W1_b, (0, VPAD - VOCAB)).reshape(VPAD, 1)
    out_t = _tc_project_t(emb, W1_w, bpad)
    return out_t.T


# speedup vs baseline: 1.1415x; 1.0033x over previous
import functools

import jax
import jax.numpy as jnp
import numpy as np
from jax import lax
from jax.experimental import pallas as pl
from jax.experimental.pallas import tpu as pltpu
from jax.experimental.pallas import tpu_sc as plsc

VOCAB = 100000
EMB = 128
BATCH = 1024
CTX = 20

NC, NS = 2, 16
NW = NC * NS
N_IDX = BATCH * CTX
B_PER_W = N_IDX // NW
ROWS_PER_W = BATCH // NW
ROWS_PER_C = BATCH // NC
IDX_CHUNK = 128
CHUNKS = B_PER_W // IDX_CHUNK
SEG_ROWS = 8

VT = 1024

_SEG_NP = np.zeros((NW, SEG_ROWS, IDX_CHUNK), np.int32)
_seg_flat = ((np.arange(N_IDX) // CTX) % ROWS_PER_C).astype(np.int32)
_SEG_NP[:, :CHUNKS, :] = _seg_flat.reshape(NW, CHUNKS, IDX_CHUNK)


def _sc_embed(table, idx_flat, seg_map, zeros_blk):
    mesh = plsc.VectorSubcoreMesh(core_axis_name="c", subcore_axis_name="s")

    @functools.partial(
        pl.kernel,
        out_type=jax.ShapeDtypeStruct((BATCH, EMB), jnp.float32),
        mesh=mesh,
        scratch_types=[
            pltpu.VMEM((B_PER_W,), jnp.int32),
            pltpu.VMEM((SEG_ROWS, IDX_CHUNK), jnp.int32),
            pltpu.VMEM((B_PER_W, EMB), jnp.float32),
            pltpu.VMEM_SHARED((ROWS_PER_C, EMB), jnp.float32),
            pltpu.SemaphoreType.DMA,
        ],
    )
    def embed_kernel(table_hbm, idx_hbm, seg_hbm, zeros_hbm, out_hbm,
                     idx_v, seg_v, rows_v, emb_sh, sem):
        cid = lax.axis_index("c")
        sid = lax.axis_index("s")
        wid = cid * NS + sid
        pltpu.sync_copy(idx_hbm.at[pl.ds(wid * B_PER_W, B_PER_W)], idx_v)
        pltpu.sync_copy(seg_hbm.at[wid], seg_v)

        @pl.when(sid == 0)
        def _():
            pltpu.sync_copy(zeros_hbm, emb_sh)

        copies = []
        for j in range(CHUNKS):
            copies.append(
                pltpu.async_copy(
                    table_hbm.at[idx_v.at[pl.ds(j * IDX_CHUNK, IDX_CHUNK)]],
                    rows_v.at[pl.ds(j * IDX_CHUNK, IDX_CHUNK)],
                    sem,
                )
            )
        plsc.subcore_barrier()
        for j, c in enumerate(copies):
            c.wait()
            pltpu.sync_copy(
                rows_v.at[pl.ds(j * IDX_CHUNK, IDX_CHUNK)],
                emb_sh.at[seg_v.at[j]],
                add=True,
            )
        plsc.subcore_barrier()
        pltpu.sync_copy(
            emb_sh.at[pl.ds(sid * ROWS_PER_W, ROWS_PER_W)],
            out_hbm.at[pl.ds(wid * ROWS_PER_W, ROWS_PER_W)],
        )

    return embed_kernel(table, idx_flat, seg_map, zeros_blk)


NBUF = 8
GROUP = NBUF * VT
MACRO = -(-VOCAB // GROUP)
VPAD = MACRO * GROUP
LAST_BASE = (MACRO - 1) * GROUP
LAST_ROWS = VOCAB - LAST_BASE


def _tc_project_t(emb, w, bpad):
    def body(emb_ref, w_ref, b_ref, o_hbm, ebf_ref, bufs, sems):
        m = pl.program_id(0)

        @pl.when(m == 0)
        def _():
            ebf_ref[...] = emb_ref[...].astype(jnp.bfloat16)

        def out_copy(macro_idx, j, rows=VT):
            return pltpu.make_async_copy(
                bufs.at[j, pl.ds(0, rows)],
                o_hbm.at[pl.ds(macro_idx * GROUP + j * VT, rows)],
                sems.at[j],
            )

        def last_valid(j):
            return min(VT, max(0, VOCAB - (LAST_BASE + j * VT)))

        for j in range(NBUF):
            @pl.when(m >= 1)
            def _(j=j):
                out_copy(m - 1, j).wait()

            wt = w_ref[pl.ds(j * VT, VT), :].astype(jnp.bfloat16)
            acc = lax.dot_general(
                wt,
                ebf_ref[...],
                dimension_numbers=(((1,), (1,)), ((), ())),
                preferred_element_type=jnp.float32,
            )
            bufs[j] = acc + b_ref[pl.ds(j * VT, VT), :]
            if last_valid(j) == VT:
                out_copy(m, j).start()
            elif last_valid(j) > 0:
                @pl.when(m < MACRO - 1)
                def _(j=j):
                    out_copy(m, j).start()

                @pl.when(m == MACRO - 1)
                def _(j=j):
                    out_copy(MACRO - 1, j, rows=last_valid(j)).start()
            else:
                @pl.when(m < MACRO - 1)
                def _(j=j):
                    out_copy(m, j).start()

        @pl.when(m == MACRO - 1)
        def _():
            for j in range(NBUF):
                if last_valid(j) > 0:
                    out_copy(MACRO - 1, j, rows=last_valid(j)).wait()

    return pl.pallas_call(
        body,
        grid=(MACRO,),
        in_specs=[
            pl.BlockSpec((BATCH, EMB), lambda i: (0, 0)),
            pl.BlockSpec((GROUP, EMB), lambda i: (i, 0)),
            pl.BlockSpec((GROUP, 1), lambda i: (i, 0)),
        ],
        out_specs=pl.BlockSpec(memory_space=pl.ANY),
        out_shape=jax.ShapeDtypeStruct((VOCAB, BATCH), jnp.float32),
        scratch_shapes=[
            pltpu.VMEM((BATCH, EMB), jnp.bfloat16),
            pltpu.VMEM((NBUF, VT, BATCH), jnp.float32),
            pltpu.SemaphoreType.DMA((NBUF,)),
        ],
        compiler_params=pltpu.CompilerParams(
            dimension_semantics=("arbitrary",),
        ),
    )(emb, w, bpad)


def kernel(X, W_emb, W1_w, W1_b):
    seg_map = jnp.asarray(_SEG_NP)
    zeros_blk = jnp.zeros((ROWS_PER_C, EMB), jnp.float32)
    emb = _sc_embed(W_emb, X.reshape(N_IDX), seg_map, zeros_blk)
    bpad = jnp.pad(W1_b, (0, VPAD - VOCAB)).reshape(VPAD, 1)
    out_t = _tc_project_t(emb, W1_w, bpad)
    return out_t.T
